# Initial kernel scaffold; baseline (speedup 1.0000x reference)
import jax
import jax.numpy as jnp
from jax.experimental import pallas as pl


def kernel(x, edge_index, W_self1, W_neigh1, b1, W_self2, W_neigh2, b2, W_cls, b_cls):
    raise NotImplementedError("write your pallas kernel here")


# speedup vs baseline: 6.8974x; 6.8974x over previous
import functools

import jax
import jax.numpy as jnp
from jax import lax
from jax.experimental import pallas as pl
from jax.experimental.pallas import tpu as pltpu
from jax.experimental.pallas import tpu_sc as plsc

N_NODES = 10000
N_EDGES = 320000
FDIM = 128
NC = 2
NS = 16
NW = NC * NS
EDGES_PER_TILE = N_EDGES // NW
CHUNK = 80
NCHUNK = EDGES_PER_TILE // CHUNK
NPAD = 10112
ROWS_PER_TILE = NPAD // NS
DEGW = 128


def _sc_agg_body(x_hbm, src_hbm, dst_hbm, z_hbm, acc_out,
                 src_v, dst_v, gbuf, acc_sh, sem):
    c = lax.axis_index("c")
    s = lax.axis_index("s")
    t = c * NS + s

    pltpu.sync_copy(src_hbm.at[t], src_v)
    pltpu.sync_copy(dst_hbm.at[t], dst_v)
    rs = pl.ds(s * ROWS_PER_TILE, ROWS_PER_TILE)
    pltpu.sync_copy(z_hbm, acc_sh.at[rs])
    plsc.subcore_barrier()

    def body(ci, carry):
        pltpu.async_copy(x_hbm.at[src_v.at[ci]], gbuf, sem).wait()
        pltpu.sync_copy(gbuf, acc_sh.at[dst_v.at[ci]], add=True)
        return carry

    lax.fori_loop(0, NCHUNK, body, 0)
    plsc.subcore_barrier()
    pltpu.sync_copy(acc_sh.at[rs], acc_out.at[c].at[rs])


_sc_agg = pl.kernel(
    _sc_agg_body,
    out_type=jax.ShapeDtypeStruct((NC, NPAD, FDIM), jnp.float32),
    mesh=plsc.VectorSubcoreMesh(core_axis_name="c", subcore_axis_name="s"),
    scratch_types=[
        pltpu.VMEM((NCHUNK, CHUNK), jnp.int32),
        pltpu.VMEM((NCHUNK, CHUNK), jnp.int32),
        pltpu.VMEM((CHUNK, FDIM), jnp.float32),
        pltpu.VMEM_SHARED((NPAD, FDIM), jnp.float32),
        pltpu.SemaphoreType.DMA,
    ],
)


def _sc_deg_body(dst_hbm, zd_hbm, ones_hbm, deg_out, dst_v, ones_v, deg_sh):
    c = lax.axis_index("c")
    s = lax.axis_index("s")
    t = c * NS + s

    pltpu.sync_copy(dst_hbm.at[t], dst_v)
    pltpu.sync_copy(ones_hbm, ones_v)
    rs = pl.ds(s * ROWS_PER_TILE, ROWS_PER_TILE)
    pltpu.sync_copy(zd_hbm, deg_sh.at[rs])
    plsc.subcore_barrier()

    def body(ci, carry):
        pltpu.sync_copy(ones_v, deg_sh.at[dst_v.at[ci]], add=True)
        return carry

    lax.fori_loop(0, NCHUNK, body, 0)
    plsc.subcore_barrier()
    pltpu.sync_copy(deg_sh.at[rs], deg_out.at[c].at[rs])


_sc_deg = pl.kernel(
    _sc_deg_body,
    out_type=jax.ShapeDtypeStruct((NC, NPAD, DEGW), jnp.float32),
    mesh=plsc.VectorSubcoreMesh(core_axis_name="c", subcore_axis_name="s"),
    scratch_types=[
        pltpu.VMEM((NCHUNK, CHUNK), jnp.int32),
        pltpu.VMEM((CHUNK, DEGW), jnp.float32),
        pltpu.VMEM_SHARED((NPAD, DEGW), jnp.float32),
    ],
)


def _tc_layer_body(x_ref, acc_ref, deg_ref, ws_ref, wn_ref, b_ref, o_ref):
    deg = deg_ref[0, :, 0:1] + deg_ref[1, :, 0:1]
    mean = (acc_ref[0] + acc_ref[1]) / jnp.maximum(deg, 1.0)
    h = (jnp.dot(x_ref[...], ws_ref[...], preferred_element_type=jnp.float32)
         + jnp.dot(mean, wn_ref[...], preferred_element_type=jnp.float32)
         + b_ref[...])
    o_ref[...] = jnp.maximum(h, 0.0)


def _tc_final_body(x_ref, acc_ref, deg_ref, ws_ref, wn_ref, b_ref,
                   wc_ref, bc_ref, o_ref, accum):
    i = pl.program_id(0)
    deg = deg_ref[0, :, 0:1] + deg_ref[1, :, 0:1]
    mean = (acc_ref[0] + acc_ref[1]) / jnp.maximum(deg, 1.0)
    h = (jnp.dot(x_ref[...], ws_ref[...], preferred_element_type=jnp.float32)
         + jnp.dot(mean, wn_ref[...], preferred_element_type=jnp.float32)
         + b_ref[...])
    h = jnp.maximum(h, 0.0)

    @pl.when(i == 0)
    def _():
        accum[...] = jnp.zeros_like(accum)

    accum[...] += jnp.sum(h, axis=0, keepdims=True)

    @pl.when(i == pl.num_programs(0) - 1)
    def _():
        hg = accum[...] * (1.0 / N_NODES)
        o_ref[...] = (jnp.dot(hg, wc_ref[...], preferred_element_type=jnp.float32)
                      + bc_ref[...])


_BLK = 1000
_GRID = N_NODES // _BLK

_COMMON_SPECS = [
    pl.BlockSpec((_BLK, FDIM), lambda i: (i, 0)),
    pl.BlockSpec((NC, _BLK, FDIM), lambda i: (0, i, 0)),
    pl.BlockSpec((NC, _BLK, DEGW), lambda i: (0, i, 0)),
    pl.BlockSpec((FDIM, FDIM), lambda i: (0, 0)),
    pl.BlockSpec((FDIM, FDIM), lambda i: (0, 0)),
    pl.BlockSpec((1, FDIM), lambda i: (0, 0)),
]


def _tc_layer(x, acc, deg, ws, wn, b):
    return pl.pallas_call(
        _tc_layer_body,
        grid=(_GRID,),
        in_specs=list(_COMMON_SPECS),
        out_specs=pl.BlockSpec((_BLK, FDIM), lambda i: (i, 0)),
        out_shape=jax.ShapeDtypeStruct((N_NODES, FDIM), jnp.float32),
    )(x, acc, deg, ws, wn, b)


def _tc_final(x, acc, deg, ws, wn, b, wc, bc):
    return pl.pallas_call(
        _tc_final_body,
        grid=(_GRID,),
        in_specs=list(_COMMON_SPECS) + [
            pl.BlockSpec((FDIM, FDIM), lambda i: (0, 0)),
            pl.BlockSpec((1, FDIM), lambda i: (0, 0)),
        ],
        out_specs=pl.BlockSpec((1, FDIM), lambda i: (0, 0)),
        out_shape=jax.ShapeDtypeStruct((1, FDIM), jnp.float32),
        scratch_shapes=[pltpu.VMEM((1, FDIM), jnp.float32)],
    )(x, acc, deg, ws, wn, b, wc, bc)


def kernel(x, edge_index, W_self1, W_neigh1, b1, W_self2, W_neigh2, b2,
           W_cls, b_cls):
    src = edge_index[0].reshape(NW, NCHUNK, CHUNK)
    dst = edge_index[1].reshape(NW, NCHUNK, CHUNK)
    z = jnp.zeros((ROWS_PER_TILE, FDIM), jnp.float32)
    ones = jnp.ones((CHUNK, DEGW), jnp.float32)
    b1r = b1.reshape(1, FDIM)
    b2r = b2.reshape(1, FDIM)
    wc = jnp.zeros((FDIM, FDIM), jnp.float32).at[:, :W_cls.shape[1]].set(W_cls)
    bc = jnp.zeros((1, FDIM), jnp.float32).at[0, :b_cls.shape[0]].set(b_cls)

    deg = _sc_deg(dst, z, ones)
    acc1 = _sc_agg(x, src, dst, z)
    h1 = _tc_layer(x, acc1, deg, W_self1, W_neigh1, b1r)
    acc2 = _sc_agg(h1, src, dst, z)
    out = _tc_final(h1, acc2, deg, W_self2, W_neigh2, b2r, wc, bc)
    return out[:, :W_cls.shape[1]]


# speedup vs baseline: 9.6194x; 1.3946x over previous
import jax
import jax.numpy as jnp
from jax import lax
from jax.experimental import pallas as pl
from jax.experimental.pallas import tpu as pltpu
from jax.experimental.pallas import tpu_sc as plsc

N_NODES = 10000
N_EDGES = 320000
FDIM = 128
NC = 2
NS = 16
NW = NC * NS
EDGES_PER_TILE = N_EDGES // NW
CHUNK = 125
NCHUNK = EDGES_PER_TILE // CHUNK
NPAD = 10112
ROWS_PER_TILE = NPAD // NS
DEGW = 16
DEG_FIRE = 8


IDXBLK = 40


def _sc_agg_body(x_hbm, src_hbm, dst_hbm, z_hbm, acc_out,
                 src_v, dst_v, g0, g1, acc_sh, sem0, sem1):
    c = lax.axis_index("c")
    s = lax.axis_index("s")
    t = c * NS + s

    rs = pl.ds(s * ROWS_PER_TILE, ROWS_PER_TILE)
    pltpu.sync_copy(z_hbm, acc_sh.at[rs])
    plsc.subcore_barrier()

    for blk in range(NCHUNK // IDXBLK):
        bs = pl.ds(blk * IDXBLK, IDXBLK)
        pltpu.sync_copy(src_hbm.at[t].at[bs], src_v)
        pltpu.sync_copy(dst_hbm.at[t].at[bs], dst_v)
        pltpu.async_copy(x_hbm.at[src_v.at[0]], g0, sem0)

        def body(i, carry):
            a = 2 * i
            b = a + 1
            nxt = a + 2
            pltpu.make_async_copy(x_hbm.at[src_v.at[a]], g0, sem0).wait()
            pltpu.async_copy(x_hbm.at[src_v.at[b]], g1, sem1)
            pltpu.sync_copy(g0, acc_sh.at[dst_v.at[a]], add=True)
            pltpu.make_async_copy(x_hbm.at[src_v.at[b]], g1, sem1).wait()

            @pl.when(nxt < IDXBLK)
            def _():
                pltpu.async_copy(x_hbm.at[src_v.at[nxt]], g0, sem0)

            pltpu.sync_copy(g1, acc_sh.at[dst_v.at[b]], add=True)
            return carry

        lax.fori_loop(0, IDXBLK // 2, body, 0)

    plsc.subcore_barrier()
    pltpu.sync_copy(acc_sh.at[rs], acc_out.at[c].at[rs])


_sc_agg = pl.kernel(
    _sc_agg_body,
    out_type=jax.ShapeDtypeStruct((NC, NPAD, FDIM), jnp.float32),
    mesh=plsc.VectorSubcoreMesh(core_axis_name="c", subcore_axis_name="s"),
    scratch_types=[
        pltpu.VMEM((IDXBLK, CHUNK), jnp.int32),
        pltpu.VMEM((IDXBLK, CHUNK), jnp.int32),
        pltpu.VMEM((CHUNK, FDIM), jnp.float32),
        pltpu.VMEM((CHUNK, FDIM), jnp.float32),
        pltpu.VMEM_SHARED((NPAD, FDIM), jnp.float32),
        pltpu.SemaphoreType.DMA,
        pltpu.SemaphoreType.DMA,
    ],
)


def _sc_deg_body(dst8_hbm, zd_hbm, ones_hbm, deg_out, dst_v, ones_v, deg_sh,
                 sem):
    c = lax.axis_index("c")
    s = lax.axis_index("s")
    t = c * NS + s

    pltpu.sync_copy(dst8_hbm.at[t], dst_v)
    pltpu.sync_copy(ones_hbm, ones_v)
    rs = pl.ds(s * ROWS_PER_TILE, ROWS_PER_TILE)
    pltpu.sync_copy(zd_hbm, deg_sh.at[rs])
    plsc.subcore_barrier()

    def body(i, carry):
        for j in range(DEG_FIRE):
            pltpu.async_copy(
                ones_v, deg_sh.at[dst_v.at[i * DEG_FIRE + j]], sem, add=True)
        for j in range(DEG_FIRE):
            pltpu.make_async_copy(
                ones_v, deg_sh.at[dst_v.at[i * DEG_FIRE + j]], sem).wait()
        return carry

    lax.fori_loop(0, NCHUNK // DEG_FIRE, body, 0)
    plsc.subcore_barrier()
    pltpu.sync_copy(deg_sh.at[rs], deg_out.at[c].at[rs])


_sc_deg = pl.kernel(
    _sc_deg_body,
    out_type=jax.ShapeDtypeStruct((NC, NPAD, DEGW), jnp.float32),
    mesh=plsc.VectorSubcoreMesh(core_axis_name="c", subcore_axis_name="s"),
    scratch_types=[
        pltpu.VMEM((NCHUNK, CHUNK), jnp.int32),
        pltpu.VMEM((CHUNK, DEGW), jnp.float32),
        pltpu.VMEM_SHARED((NPAD, DEGW), jnp.float32),
        pltpu.SemaphoreType.DMA,
    ],
)


def _tc_layer_body(x_ref, acc_ref, deg_ref, ws_ref, wn_ref, b_ref, o_ref):
    deg = deg_ref[0, :, 0:1] + deg_ref[1, :, 0:1]
    mean = (acc_ref[0] + acc_ref[1]) / jnp.maximum(deg, 1.0)
    h = (jnp.dot(x_ref[...], ws_ref[...], preferred_element_type=jnp.float32)
         + jnp.dot(mean, wn_ref[...], preferred_element_type=jnp.float32)
         + b_ref[...])
    o_ref[...] = jnp.maximum(h, 0.0)


def _tc_final_body(x_ref, acc_ref, deg_ref, ws_ref, wn_ref, b_ref,
                   wc_ref, bc_ref, o_ref, accum):
    i = pl.program_id(0)
    deg = deg_ref[0, :, 0:1] + deg_ref[1, :, 0:1]
    mean = (acc_ref[0] + acc_ref[1]) / jnp.maximum(deg, 1.0)
    h = (jnp.dot(x_ref[...], ws_ref[...], preferred_element_type=jnp.float32)
         + jnp.dot(mean, wn_ref[...], preferred_element_type=jnp.float32)
         + b_ref[...])
    h = jnp.maximum(h, 0.0)

    @pl.when(i == 0)
    def _():
        accum[...] = jnp.zeros_like(accum)

    accum[...] += jnp.sum(h, axis=0, keepdims=True)

    @pl.when(i == pl.num_programs(0) - 1)
    def _():
        hg = accum[...] * (1.0 / N_NODES)
        o_ref[...] = (jnp.dot(hg, wc_ref[...], preferred_element_type=jnp.float32)
                      + bc_ref[...])


_BLK = 1000
_GRID = N_NODES // _BLK

_COMMON_SPECS = [
    pl.BlockSpec((_BLK, FDIM), lambda i: (i, 0)),
    pl.BlockSpec((NC, _BLK, FDIM), lambda i: (0, i, 0)),
    pl.BlockSpec((NC, _BLK, DEGW), lambda i: (0, i, 0)),
    pl.BlockSpec((FDIM, FDIM), lambda i: (0, 0)),
    pl.BlockSpec((FDIM, FDIM), lambda i: (0, 0)),
    pl.BlockSpec((1, FDIM), lambda i: (0, 0)),
]


def _tc_layer(x, acc, deg, ws, wn, b):
    return pl.pallas_call(
        _tc_layer_body,
        grid=(_GRID,),
        in_specs=list(_COMMON_SPECS),
        out_specs=pl.BlockSpec((_BLK, FDIM), lambda i: (i, 0)),
        out_shape=jax.ShapeDtypeStruct((N_NODES, FDIM), jnp.float32),
    )(x, acc, deg, ws, wn, b)


def _tc_final(x, acc, deg, ws, wn, b, wc, bc):
    return pl.pallas_call(
        _tc_final_body,
        grid=(_GRID,),
        in_specs=list(_COMMON_SPECS) + [
            pl.BlockSpec((FDIM, FDIM), lambda i: (0, 0)),
            pl.BlockSpec((1, FDIM), lambda i: (0, 0)),
        ],
        out_specs=pl.BlockSpec((1, FDIM), lambda i: (0, 0)),
        out_shape=jax.ShapeDtypeStruct((1, FDIM), jnp.float32),
        scratch_shapes=[pltpu.VMEM((1, FDIM), jnp.float32)],
    )(x, acc, deg, ws, wn, b, wc, bc)


def kernel(x, edge_index, W_self1, W_neigh1, b1, W_self2, W_neigh2, b2,
           W_cls, b_cls):
    src = edge_index[0].reshape(NW, NCHUNK, CHUNK)
    dst = edge_index[1].reshape(NW, NCHUNK, CHUNK)
    dst8 = dst * 8
    z = jnp.zeros((ROWS_PER_TILE, FDIM), jnp.float32)
    zd = jnp.zeros((ROWS_PER_TILE, DEGW), jnp.float32)
    ones = jnp.ones((CHUNK, DEGW), jnp.float32)
    b1r = b1.reshape(1, FDIM)
    b2r = b2.reshape(1, FDIM)
    wc = jnp.zeros((FDIM, FDIM), jnp.float32).at[:, :W_cls.shape[1]].set(W_cls)
    bc = jnp.zeros((1, FDIM), jnp.float32).at[0, :b_cls.shape[0]].set(b_cls)

    deg = _sc_deg(dst8, zd, ones)
    acc1 = _sc_agg(x, src, dst, z)
    h1 = _tc_layer(x, acc1, deg, W_self1, W_neigh1, b1r)
    acc2 = _sc_agg(h1, src, dst, z)
    out = _tc_final(h1, acc2, deg, W_self2, W_neigh2, b2r, wc, bc)
    return out[:, :W_cls.shape[1]]
